# initial kernel scaffold (unmeasured)
import jax
import jax.numpy as jnp
from jax import lax
from jax.experimental import pallas as pl
from jax.experimental.pallas import tpu as pltpu

N_DEV = 4
F8 = jnp.float8_e5m2
NC = 2048


def kernel(x, w_mat, scale_x, scale_w):
    m_total, k_shard = x.shape
    k_total, n_total = w_mat.shape
    m_per = m_total // N_DEV
    n_tiles = n_total // NC
    n_steps = N_DEV * n_tiles
    w_is_f8 = w_mat.dtype == F8

    def body(scale_x_ref, scale_w_ref, x_ref, w_hbm, out_ref,
             x8_ref, comm_ref, wbuf, send_sems, recv_sems, wsems):
        my = lax.axis_index("i")

        barrier = pltpu.get_barrier_semaphore()
        for d in (1, 2, 3):
            peer = lax.rem(my + d, N_DEV)
            pl.semaphore_signal(
                barrier, inc=1,
                device_id=(peer,), device_id_type=pl.DeviceIdType.MESH,
            )
        pl.semaphore_wait(barrier, 3)

        x8_ref[...] = x_ref[...].astype(F8)

        sends = []
        for d in (1, 3, 2):
            peer = lax.rem(my + d, N_DEV)
            rdma = pltpu.make_async_remote_copy(
                src_ref=x8_ref.at[pl.ds(peer * m_per, m_per), :],
                dst_ref=comm_ref.at[my],
                send_sem=send_sems.at[peer],
                recv_sem=recv_sems.at[my],
                device_id=(peer,),
                device_id_type=pl.DeviceIdType.MESH,
            )
            rdma.start()
            sends.append(rdma)

        def wait_recv_from(p):
            pltpu.make_async_remote_copy(
                src_ref=x8_ref.at[pl.ds(0, m_per), :],
                dst_ref=comm_ref.at[p],
                send_sem=send_sems.at[p],
                recv_sem=recv_sems.at[p],
                device_id=(my,),
                device_id_type=pl.DeviceIdType.MESH,
            ).wait_recv()

        p_list = [my] + [lax.rem(my + d, N_DEV) for d in (1, 3, 2)]

        def w_dma(step, slot):
            p = p_list[step // n_tiles]
            j = step % n_tiles
            return pltpu.make_async_copy(
                w_hbm.at[pl.ds(p * m_per, m_per), pl.ds(j * NC, NC)],
                wbuf.at[slot],
                wsems.at[slot],
            )

        sc = scale_x_ref[0] * scale_w_ref[0]

        w_dma(0, 0).start()
        for s in range(n_steps):
            slot = s % 2
            if s + 1 < n_steps:
                w_dma(s + 1, (s + 1) % 2).start()
            p_idx, j = s // n_tiles, s % n_tiles
            if j == 0 and p_idx > 0:
                wait_recv_from(p_list[p_idx])
            w_dma(s, slot).wait()

            if p_idx == 0:
                lhs = x8_ref[pl.ds(my * m_per, m_per), :]
            else:
                lhs = comm_ref[p_list[p_idx]]
            rhs = wbuf[slot] if w_is_f8 else wbuf[slot].astype(F8)
            part = lax.dot_general(
                lhs, rhs, (((1,), (0,)), ((), ())),
                preferred_element_type=jnp.float32,
            )
            nd = pl.ds(j * NC, NC)
            if p_idx == 0:
                out_ref[:, nd] = part
            elif p_idx == N_DEV - 1:
                out_ref[:, nd] = jnp.maximum((out_ref[:, nd] + part) * sc, 0.0)
            else:
                out_ref[:, nd] += part

        for rdma in sends:
            rdma.wait_send()

    return pl.pallas_call(
        body,
        out_shape=jax.ShapeDtypeStruct((m_per, n_total), jnp.float32),
        in_specs=[
            pl.BlockSpec(memory_space=pltpu.SMEM),
            pl.BlockSpec(memory_space=pltpu.SMEM),
            pl.BlockSpec(memory_space=pltpu.VMEM),
            pl.BlockSpec(memory_space=pltpu.ANY),
        ],
        out_specs=pl.BlockSpec(memory_space=pltpu.VMEM),
        scratch_shapes=[
            pltpu.VMEM((m_total, k_shard), F8),
            pltpu.VMEM((N_DEV, m_per, k_shard), F8),
            pltpu.VMEM((2, m_per, NC), w_mat.dtype),
            pltpu.SemaphoreType.DMA((N_DEV,)),
            pltpu.SemaphoreType.DMA((N_DEV,)),
            pltpu.SemaphoreType.DMA((2,)),
        ],
        compiler_params=pltpu.CompilerParams(collective_id=0),
    )(scale_x, scale_w, x, w_mat)


# baseline (device time: 103962 ns/iter reference)
import jax
import jax.numpy as jnp
from jax import lax
from jax.experimental import pallas as pl
from jax.experimental.pallas import tpu as pltpu

N_DEV = 4
F8 = jnp.float8_e5m2
NC = 1024


def kernel(x, w_mat, scale_x, scale_w):
    m_total, k_shard = x.shape
    k_total, n_total = w_mat.shape
    m_per = m_total // N_DEV
    n_tiles = n_total // NC
    n_steps = N_DEV * n_tiles
    w_is_f8 = w_mat.dtype == F8
    x_is_f8 = x.dtype == F8

    def body(scale_x_ref, scale_w_ref, x_hbm, w_hbm, out_ref,
             sbuf, send8, comm_ref, wbuf, xsems, send_sems, recv_sems,
             wsems):
        my = lax.axis_index("i")

        barrier = pltpu.get_barrier_semaphore()
        for d in (1, 2, 3):
            peer = lax.rem(my + d, N_DEV)
            pl.semaphore_signal(
                barrier, inc=1,
                device_id=(peer,), device_id_type=pl.DeviceIdType.MESH,
            )
        pl.semaphore_wait(barrier, 3)

        q_order = [lax.rem(my + d, N_DEV) for d in (1, 3, 2, 0)]

        def x_dma(qi, slot):
            return pltpu.make_async_copy(
                x_hbm.at[pl.ds(q_order[qi] * m_per, m_per), :],
                sbuf.at[slot],
                xsems.at[slot],
            )

        sends = []
        x_dma(0, 0).start()
        x_dma(1, 1).start()
        for qi in range(N_DEV):
            slot = qi % 2
            x_dma(qi, slot).wait()
            q = q_order[qi]
            send8[q] = sbuf[slot].astype(F8)
            if qi + 2 < N_DEV:
                x_dma(qi + 2, slot).start()
            if qi < 3:
                rdma = pltpu.make_async_remote_copy(
                    src_ref=send8.at[q],
                    dst_ref=comm_ref.at[my],
                    send_sem=send_sems.at[q],
                    recv_sem=recv_sems.at[my],
                    device_id=(q,),
                    device_id_type=pl.DeviceIdType.MESH,
                )
                rdma.start()
                sends.append(rdma)

        def wait_recv_from(p):
            pltpu.make_async_remote_copy(
                src_ref=send8.at[p],
                dst_ref=comm_ref.at[p],
                send_sem=send_sems.at[p],
                recv_sem=recv_sems.at[p],
                device_id=(my,),
                device_id_type=pl.DeviceIdType.MESH,
            ).wait_recv()

        p_list = [my] + [lax.rem(my + d, N_DEV) for d in (1, 3, 2)]

        def w_dma(step, slot):
            p = p_list[step // n_tiles]
            j = step % n_tiles
            return pltpu.make_async_copy(
                w_hbm.at[pl.ds(p * m_per, m_per), pl.ds(j * NC, NC)],
                wbuf.at[slot],
                wsems.at[slot],
            )

        sc = scale_x_ref[0] * scale_w_ref[0]

        w_dma(0, 0).start()
        for s in range(n_steps):
            slot = s % 2
            if s + 1 < n_steps:
                w_dma(s + 1, (s + 1) % 2).start()
            p_idx, j = s // n_tiles, s % n_tiles
            if j == 0 and p_idx > 0:
                wait_recv_from(p_list[p_idx])
            w_dma(s, slot).wait()

            if p_idx == 0:
                lhs = send8[my]
            else:
                lhs = comm_ref[p_list[p_idx]]
            rhs = wbuf[slot] if w_is_f8 else wbuf[slot].astype(F8)
            part = lax.dot_general(
                lhs, rhs, (((1,), (0,)), ((), ())),
                preferred_element_type=jnp.float32,
            )
            nd = pl.ds(j * NC, NC)
            if p_idx == 0:
                out_ref[:, nd] = part
            elif p_idx == N_DEV - 1:
                out_ref[:, nd] = jnp.maximum((out_ref[:, nd] + part) * sc, 0.0)
            else:
                out_ref[:, nd] += part

        for rdma in sends:
            rdma.wait_send()

    x_stage_dtype = x.dtype
    return pl.pallas_call(
        body,
        out_shape=jax.ShapeDtypeStruct((m_per, n_total), jnp.float32),
        in_specs=[
            pl.BlockSpec(memory_space=pltpu.SMEM),
            pl.BlockSpec(memory_space=pltpu.SMEM),
            pl.BlockSpec(memory_space=pl.ANY),
            pl.BlockSpec(memory_space=pl.ANY),
        ],
        out_specs=pl.BlockSpec(memory_space=pltpu.VMEM),
        scratch_shapes=[
            pltpu.VMEM((2, m_per, k_shard), x_stage_dtype),
            pltpu.VMEM((N_DEV, m_per, k_shard), F8),
            pltpu.VMEM((N_DEV, m_per, k_shard), F8),
            pltpu.VMEM((2, m_per, NC), w_mat.dtype),
            pltpu.SemaphoreType.DMA((2,)),
            pltpu.SemaphoreType.DMA((N_DEV,)),
            pltpu.SemaphoreType.DMA((N_DEV,)),
            pltpu.SemaphoreType.DMA((2,)),
        ],
        compiler_params=pltpu.CompilerParams(
            collective_id=0,
            vmem_limit_bytes=110 * 1024 * 1024,
        ),
    )(scale_x, scale_w, x, w_mat)


# device time: 97933 ns/iter; 1.0616x vs baseline; 1.0616x over previous
import jax
import jax.numpy as jnp
from jax import lax
from jax.experimental import pallas as pl
from jax.experimental.pallas import tpu as pltpu

N_DEV = 4
F8 = jnp.float8_e5m2
NC = 1024
W_SLOTS = 4


def _cast_kernel(x):
    if x.dtype == F8:
        return x

    def body(x_ref, o_ref):
        o_ref[...] = x_ref[...].astype(F8)

    return pl.pallas_call(
        body,
        out_shape=jax.ShapeDtypeStruct(x.shape, F8),
        in_specs=[pl.BlockSpec(memory_space=pltpu.VMEM)],
        out_specs=pl.BlockSpec(memory_space=pltpu.VMEM),
        compiler_params=pltpu.CompilerParams(
            vmem_limit_bytes=48 * 1024 * 1024,
        ),
    )(x)


def kernel(x, w_mat, scale_x, scale_w):
    m_total, k_shard = x.shape
    k_total, n_total = w_mat.shape
    m_per = m_total // N_DEV
    n_tiles = n_total // NC
    n_steps = N_DEV * n_tiles
    w_is_f8 = w_mat.dtype == F8

    x8 = _cast_kernel(x)

    def body(scale_x_ref, scale_w_ref, x_ref, w_hbm, out_ref,
             comm_ref, wbuf, send_sems, recv_sems, wsems):
        my = lax.axis_index("i")

        barrier = pltpu.get_barrier_semaphore()
        for d in (1, 2, 3):
            peer = lax.rem(my + d, N_DEV)
            pl.semaphore_signal(
                barrier, inc=1,
                device_id=(peer,), device_id_type=pl.DeviceIdType.MESH,
            )
        pl.semaphore_wait(barrier, 3)

        sends = []
        for d in (1, 3, 2):
            peer = lax.rem(my + d, N_DEV)
            rdma = pltpu.make_async_remote_copy(
                src_ref=x_ref.at[pl.ds(peer * m_per, m_per), :],
                dst_ref=comm_ref.at[my],
                send_sem=send_sems.at[peer],
                recv_sem=recv_sems.at[my],
                device_id=(peer,),
                device_id_type=pl.DeviceIdType.MESH,
            )
            rdma.start()
            sends.append(rdma)

        def wait_recv_from(p):
            pltpu.make_async_remote_copy(
                src_ref=x_ref.at[pl.ds(0, m_per), :],
                dst_ref=comm_ref.at[p],
                send_sem=send_sems.at[p],
                recv_sem=recv_sems.at[p],
                device_id=(my,),
                device_id_type=pl.DeviceIdType.MESH,
            ).wait_recv()

        p_list = [my] + [lax.rem(my + d, N_DEV) for d in (1, 3, 2)]

        def w_dma(step, slot):
            p = p_list[step // n_tiles]
            j = step % n_tiles
            return pltpu.make_async_copy(
                w_hbm.at[pl.ds(p * m_per, m_per), pl.ds(j * NC, NC)],
                wbuf.at[slot],
                wsems.at[slot],
            )

        sc = scale_x_ref[0] * scale_w_ref[0]

        for s in range(W_SLOTS - 1):
            w_dma(s, s).start()
        for s in range(n_steps):
            slot = s % W_SLOTS
            if s + W_SLOTS - 1 < n_steps:
                w_dma(s + W_SLOTS - 1, (s + W_SLOTS - 1) % W_SLOTS).start()
            p_idx, j = s // n_tiles, s % n_tiles
            if j == 0 and p_idx > 0:
                wait_recv_from(p_list[p_idx])
            w_dma(s, slot).wait()

            if p_idx == 0:
                lhs = x_ref[pl.ds(my * m_per, m_per), :]
            else:
                lhs = comm_ref[p_list[p_idx]]
            rhs = wbuf[slot] if w_is_f8 else wbuf[slot].astype(F8)
            part = lax.dot_general(
                lhs, rhs, (((1,), (0,)), ((), ())),
                preferred_element_type=jnp.float32,
            )
            nd = pl.ds(j * NC, NC)
            if p_idx == 0:
                out_ref[:, nd] = part
            elif p_idx == N_DEV - 1:
                out_ref[:, nd] = jnp.maximum((out_ref[:, nd] + part) * sc, 0.0)
            else:
                out_ref[:, nd] += part

        for rdma in sends:
            rdma.wait_send()

    return pl.pallas_call(
        body,
        out_shape=jax.ShapeDtypeStruct((m_per, n_total), jnp.float32),
        in_specs=[
            pl.BlockSpec(memory_space=pltpu.SMEM),
            pl.BlockSpec(memory_space=pltpu.SMEM),
            pl.BlockSpec(memory_space=pltpu.VMEM),
            pl.BlockSpec(memory_space=pl.ANY),
        ],
        out_specs=pl.BlockSpec(memory_space=pltpu.VMEM),
        scratch_shapes=[
            pltpu.VMEM((N_DEV, m_per, k_shard), F8),
            pltpu.VMEM((W_SLOTS, m_per, NC), w_mat.dtype),
            pltpu.SemaphoreType.DMA((N_DEV,)),
            pltpu.SemaphoreType.DMA((N_DEV,)),
            pltpu.SemaphoreType.DMA((W_SLOTS,)),
        ],
        compiler_params=pltpu.CompilerParams(
            collective_id=0,
            vmem_limit_bytes=110 * 1024 * 1024,
        ),
    )(scale_x, scale_w, x8, w_mat)
